# baseline (speedup 1.0000x reference)
import jax
import jax.numpy as jnp
from jax import lax
from jax.experimental import pallas as pl
from jax.experimental.pallas import tpu as pltpu
from jax.experimental.pallas import tpu_sc as plsc

N = 320000
C = 128
B = 16
EPS = 1e-5

NC = 2
NS = 16
NW = NC * NS
ROWS_W = N // NW
CHUNK = 400
NCHUNK = ROWS_W // CHUNK
NPAIR = NCHUNK // 2
LANES = 16
CL = C // LANES

N_SC = 192000
ROWS_WS = N_SC // NW
NCHUNK_S = ROWS_WS // CHUNK
NPAIR_S = NCHUNK_S // 2
RB = 1000
NB_SC = N_SC // RB
NB_TC = (N - N_SC) // RB

_mesh = plsc.VectorSubcoreMesh(
    core_axis_name="c", subcore_axis_name="s", num_cores=NC, num_subcores=NS
)
_sc_params = pltpu.CompilerParams(needs_layout_passes=False)


def _segment_cuts(idx_v, n):
    bvec = lax.iota(jnp.int32, LANES)
    nn = jnp.full((LANES,), n, jnp.int32)

    def step(i, lo):
        st = jnp.int32(1 << 13) >> i
        cand = lo + st
        j = jnp.minimum(cand, nn) - 1
        val = plsc.load_gather(idx_v, [j])
        ok = (cand <= nn) & (val < bvec)
        return jnp.where(ok, cand, lo)

    lo = lax.fori_loop(0, 14, step, jnp.zeros((LANES,), jnp.int32))
    cuts = [lo[b] for b in range(LANES)]
    cuts.append(jnp.int32(n))
    return cuts


def _in_copy(feat_hbm, row0, c, buf, sem):
    start = pl.multiple_of(row0 + c * CHUNK, 8)
    return pltpu.make_async_copy(feat_hbm.at[pl.ds(start, CHUNK)], buf, sem)


def _stats_body(feat_hbm, idx_hbm, out_sum, out_sq, out_cnt,
                idx_v, buf0, buf1, acc_s, acc_q, cnt_v, sem0, sem1):
    wid = lax.axis_index("s") * NC + lax.axis_index("c")
    row0 = wid * ROWS_WS
    pltpu.sync_copy(idx_hbm.at[pl.ds(row0, ROWS_WS)], idx_v)

    zero = jnp.zeros((LANES,), jnp.float32)
    for b in range(B):
        for j in range(CL):
            acc_s[b, pl.ds(j * LANES, LANES)] = zero
            acc_q[b, pl.ds(j * LANES, LANES)] = zero

    cuts = _segment_cuts(idx_v, ROWS_WS)

    def process(buf, c):
        base = c * CHUNK
        for b in range(B):
            lo = jnp.clip(cuts[b] - base, 0, CHUNK)
            hi = jnp.clip(cuts[b + 1] - base, 0, CHUNK)

            @pl.when(hi > lo)
            def _(b=b, lo=lo, hi=hi):
                def rbody(r, carry):
                    ss = list(carry[:CL])
                    qq = list(carry[CL:])
                    for j in range(CL):
                        x = buf[r, pl.ds(j * LANES, LANES)]
                        ss[j] = ss[j] + x
                        qq[j] = qq[j] + x * x
                    return tuple(ss) + tuple(qq)

                res = lax.fori_loop(lo, hi, rbody, (zero,) * (2 * CL))
                for j in range(CL):
                    sl = pl.ds(j * LANES, LANES)
                    acc_s[b, sl] = acc_s[b, sl] + res[j]
                    acc_q[b, sl] = acc_q[b, sl] + res[CL + j]

    _in_copy(feat_hbm, row0, 0, buf0, sem0).start()
    _in_copy(feat_hbm, row0, 1, buf1, sem1).start()

    def pair(p, _):
        a = 2 * p
        _in_copy(feat_hbm, row0, 0, buf0, sem0).wait()
        process(buf0, a)

        @pl.when(a + 2 < NCHUNK_S)
        def _():
            _in_copy(feat_hbm, row0, a + 2, buf0, sem0).start()

        _in_copy(feat_hbm, row0, 0, buf1, sem1).wait()
        process(buf1, a + 1)

        @pl.when(a + 3 < NCHUNK_S)
        def _():
            _in_copy(feat_hbm, row0, a + 3, buf1, sem1).start()

        return 0

    lax.fori_loop(0, NPAIR_S, pair, 0)
    if NCHUNK_S % 2:
        _in_copy(feat_hbm, row0, 0, buf0, sem0).wait()
        process(buf0, NCHUNK_S - 1)

    lanes = lax.iota(jnp.int32, LANES)
    cv = jnp.zeros((LANES,), jnp.float32)
    for b in range(B):
        cv = jnp.where(lanes == b, (cuts[b + 1] - cuts[b]).astype(jnp.float32), cv)
    cnt_v[...] = cv

    pltpu.sync_copy(acc_s, out_sum.at[wid])
    pltpu.sync_copy(acc_q, out_sq.at[wid])
    pltpu.sync_copy(cnt_v, out_cnt.at[wid])


_stats = pl.kernel(
    _stats_body,
    out_type=[
        jax.ShapeDtypeStruct((NW, B, C), jnp.float32),
        jax.ShapeDtypeStruct((NW, B, C), jnp.float32),
        jax.ShapeDtypeStruct((NW, B), jnp.float32),
    ],
    mesh=_mesh,
    scratch_types=[
        pltpu.VMEM((ROWS_WS,), jnp.int32),
        pltpu.VMEM((CHUNK, C), jnp.float32),
        pltpu.VMEM((CHUNK, C), jnp.float32),
        pltpu.VMEM((B, C), jnp.float32),
        pltpu.VMEM((B, C), jnp.float32),
        pltpu.VMEM((LANES,), jnp.float32),
        pltpu.SemaphoreType.DMA,
        pltpu.SemaphoreType.DMA,
    ],
    compiler_params=_sc_params,
)


def _stats_tc_body(x_ref, idx_ref, ps_ref, pq_ref, cnt_ref):
    i = pl.program_id(0)
    idxb = idx_ref[0, 0, :]
    onehot = (idxb[:, None]
              == lax.broadcasted_iota(jnp.int32, (RB, B), 1)).astype(jnp.float32)
    x = x_ref[...]
    dn = (((0,), (0,)), ((), ()))
    ps = lax.dot_general(onehot, x, dn, preferred_element_type=jnp.float32)
    pq = lax.dot_general(onehot, x * x, dn, preferred_element_type=jnp.float32)
    cnt = jnp.sum(onehot, axis=0, keepdims=True)

    @pl.when(i == 0)
    def _():
        ps_ref[...] = jnp.zeros_like(ps_ref)
        pq_ref[...] = jnp.zeros_like(pq_ref)
        cnt_ref[...] = jnp.zeros_like(cnt_ref)

    ps_ref[...] += ps
    pq_ref[...] += pq
    cnt_ref[...] += cnt


_stats_tc = pl.pallas_call(
    _stats_tc_body,
    grid=(NB_TC,),
    in_specs=[
        pl.BlockSpec((RB, C), lambda i: (NB_SC + i, 0)),
        pl.BlockSpec((1, 1, RB), lambda i: (NB_SC + i, 0, 0)),
    ],
    out_specs=[
        pl.BlockSpec((B, C), lambda i: (0, 0)),
        pl.BlockSpec((B, C), lambda i: (0, 0)),
        pl.BlockSpec((1, B), lambda i: (0, 0)),
    ],
    out_shape=[
        jax.ShapeDtypeStruct((B, C), jnp.float32),
        jax.ShapeDtypeStruct((B, C), jnp.float32),
        jax.ShapeDtypeStruct((1, B), jnp.float32),
    ],
)


def _combine_body(psum_ref, psq_ref, pcnt_ref, tsum_ref, tsq_ref, tcnt_ref,
                  gamma_ref, beta_ref, scale_ref, shift_ref):
    s = jnp.sum(psum_ref[...], axis=0) + tsum_ref[...]
    q = jnp.sum(psq_ref[...], axis=0) + tsq_ref[...]
    n = (jnp.sum(pcnt_ref[...], axis=0) + tcnt_ref[0, :])[:, None]
    nc = jnp.maximum(n, 1.0)
    mean = s / nc
    var = jnp.maximum(q / nc - mean * mean, 0.0)
    rstd = lax.rsqrt(var + EPS)
    scale = gamma_ref[...] * rstd
    scale_ref[...] = scale
    shift_ref[...] = beta_ref[...] - mean * scale


_combine = pl.pallas_call(
    _combine_body,
    out_shape=[
        jax.ShapeDtypeStruct((B, C), jnp.float32),
        jax.ShapeDtypeStruct((B, C), jnp.float32),
    ],
)


def _norm_body(feat_hbm, idx_hbm, scale_hbm, shift_hbm, out_hbm,
               idx_v, buf0, buf1, scale_v, shift_v, sem0, sem1, semo0, semo1):
    wid = lax.axis_index("s") * NC + lax.axis_index("c")
    row0 = wid * ROWS_W
    pltpu.sync_copy(idx_hbm.at[pl.ds(row0, ROWS_W)], idx_v)
    pltpu.sync_copy(scale_hbm, scale_v)
    pltpu.sync_copy(shift_hbm, shift_v)

    cuts = _segment_cuts(idx_v, ROWS_W)

    def out_copy(c, buf, sem):
        start = pl.multiple_of(row0 + c * CHUNK, 8)
        return pltpu.make_async_copy(buf, out_hbm.at[pl.ds(start, CHUNK)], sem)

    def process(buf, c):
        base = c * CHUNK
        for b in range(B):
            lo = jnp.clip(cuts[b] - base, 0, CHUNK)
            hi = jnp.clip(cuts[b + 1] - base, 0, CHUNK)

            @pl.when(hi > lo)
            def _(b=b, lo=lo, hi=hi):
                sc = [scale_v[b, pl.ds(j * LANES, LANES)] for j in range(CL)]
                sh = [shift_v[b, pl.ds(j * LANES, LANES)] for j in range(CL)]

                def rbody(r, _):
                    for j in range(CL):
                        sl = pl.ds(j * LANES, LANES)
                        buf[r, sl] = buf[r, sl] * sc[j] + sh[j]
                    return 0

                lax.fori_loop(lo, hi, rbody, 0)

    _in_copy(feat_hbm, row0, 0, buf0, sem0).start()
    _in_copy(feat_hbm, row0, 1, buf1, sem1).start()

    def pair(p, _):
        a = 2 * p
        _in_copy(feat_hbm, row0, 0, buf0, sem0).wait()
        process(buf0, a)
        out_copy(a, buf0, semo0).start()

        _in_copy(feat_hbm, row0, 0, buf1, sem1).wait()
        process(buf1, a + 1)
        out_copy(a + 1, buf1, semo1).start()

        out_copy(0, buf0, semo0).wait()

        @pl.when(a + 2 < NCHUNK)
        def _():
            _in_copy(feat_hbm, row0, a + 2, buf0, sem0).start()

        out_copy(0, buf1, semo1).wait()

        @pl.when(a + 3 < NCHUNK)
        def _():
            _in_copy(feat_hbm, row0, a + 3, buf1, sem1).start()

        return 0

    lax.fori_loop(0, NPAIR, pair, 0)
    if NCHUNK % 2:
        _in_copy(feat_hbm, row0, 0, buf0, sem0).wait()
        process(buf0, NCHUNK - 1)
        out_copy(NCHUNK - 1, buf0, semo0).start()
        out_copy(0, buf0, semo0).wait()


_norm = pl.kernel(
    _norm_body,
    out_type=jax.ShapeDtypeStruct((N, C), jnp.float32),
    mesh=_mesh,
    scratch_types=[
        pltpu.VMEM((ROWS_W,), jnp.int32),
        pltpu.VMEM((CHUNK, C), jnp.float32),
        pltpu.VMEM((CHUNK, C), jnp.float32),
        pltpu.VMEM((B, C), jnp.float32),
        pltpu.VMEM((B, C), jnp.float32),
        pltpu.SemaphoreType.DMA,
        pltpu.SemaphoreType.DMA,
        pltpu.SemaphoreType.DMA,
        pltpu.SemaphoreType.DMA,
    ],
    compiler_params=_sc_params,
)


def kernel(features, batch_indices, gamma, beta):
    idx = batch_indices.astype(jnp.int32)
    psum, psq, pcnt = _stats(features, idx)
    tsum, tsq, tcnt = _stats_tc(features, idx.reshape(N // RB, 1, RB))
    scale, shift = _combine(psum, psq, pcnt, tsum, tsq, tcnt,
                            gamma.reshape(1, C), beta.reshape(1, C))
    return _norm(features, idx, scale, shift)


# speedup vs baseline: 1.1018x; 1.1018x over previous
import jax
import jax.numpy as jnp
from jax import lax
from jax.experimental import pallas as pl
from jax.experimental.pallas import tpu as pltpu
from jax.experimental.pallas import tpu_sc as plsc

N = 320000
C = 128
B = 16
EPS = 1e-5

NC = 2
NS = 16
NW = NC * NS
ROWS_W = N // NW
CHUNK = 400
NCHUNK = ROWS_W // CHUNK
NPAIR = NCHUNK // 2
LANES = 16
CL = C // LANES

ROWS_WS = ROWS_W
NCHUNK_S = ROWS_WS // CHUNK
NPAIR_S = NCHUNK_S // 2

_mesh = plsc.VectorSubcoreMesh(
    core_axis_name="c", subcore_axis_name="s", num_cores=NC, num_subcores=NS
)
_sc_params = pltpu.CompilerParams(needs_layout_passes=False)


def _segment_cuts(idx_v, n):
    bvec = lax.iota(jnp.int32, LANES)
    nn = jnp.full((LANES,), n, jnp.int32)

    def step(i, lo):
        st = jnp.int32(1 << 13) >> i
        cand = lo + st
        j = jnp.minimum(cand, nn) - 1
        val = plsc.load_gather(idx_v, [j])
        ok = (cand <= nn) & (val < bvec)
        return jnp.where(ok, cand, lo)

    lo = lax.fori_loop(0, 14, step, jnp.zeros((LANES,), jnp.int32))
    cuts = [lo[b] for b in range(LANES)]
    cuts.append(jnp.int32(n))
    return cuts


def _in_copy(feat_hbm, row0, c, buf, sem):
    start = pl.multiple_of(row0 + c * CHUNK, 8)
    return pltpu.make_async_copy(feat_hbm.at[pl.ds(start, CHUNK)], buf, sem)


def _stats_body(feat_hbm, idx_hbm, out_sum, out_sq, out_cnt,
                idx_v, buf0, buf1, acc_s, acc_q, cnt_v, sem0, sem1):
    wid = lax.axis_index("s") * NC + lax.axis_index("c")
    row0 = wid * ROWS_WS
    pltpu.sync_copy(idx_hbm.at[pl.ds(row0, ROWS_WS)], idx_v)

    zero = jnp.zeros((LANES,), jnp.float32)
    for b in range(B):
        for j in range(CL):
            acc_s[b, pl.ds(j * LANES, LANES)] = zero
            acc_q[b, pl.ds(j * LANES, LANES)] = zero

    cuts = _segment_cuts(idx_v, ROWS_WS)

    def process(buf, c):
        base = c * CHUNK
        for b in range(B):
            lo = jnp.clip(cuts[b] - base, 0, CHUNK)
            hi = jnp.clip(cuts[b + 1] - base, 0, CHUNK)

            @pl.when(hi > lo)
            def _(b=b, lo=lo, hi=hi):
                def rbody(r, carry):
                    ss = list(carry[:CL])
                    qq = list(carry[CL:])
                    for j in range(CL):
                        x = buf[r, pl.ds(j * LANES, LANES)]
                        ss[j] = ss[j] + x
                        qq[j] = qq[j] + x * x
                    return tuple(ss) + tuple(qq)

                res = lax.fori_loop(lo, hi, rbody, (zero,) * (2 * CL))
                for j in range(CL):
                    sl = pl.ds(j * LANES, LANES)
                    acc_s[b, sl] = acc_s[b, sl] + res[j]
                    acc_q[b, sl] = acc_q[b, sl] + res[CL + j]

    _in_copy(feat_hbm, row0, 0, buf0, sem0).start()
    _in_copy(feat_hbm, row0, 1, buf1, sem1).start()

    def pair(p, _):
        a = 2 * p
        _in_copy(feat_hbm, row0, 0, buf0, sem0).wait()
        process(buf0, a)

        @pl.when(a + 2 < NCHUNK_S)
        def _():
            _in_copy(feat_hbm, row0, a + 2, buf0, sem0).start()

        _in_copy(feat_hbm, row0, 0, buf1, sem1).wait()
        process(buf1, a + 1)

        @pl.when(a + 3 < NCHUNK_S)
        def _():
            _in_copy(feat_hbm, row0, a + 3, buf1, sem1).start()

        return 0

    lax.fori_loop(0, NPAIR_S, pair, 0)
    if NCHUNK_S % 2:
        _in_copy(feat_hbm, row0, 0, buf0, sem0).wait()
        process(buf0, NCHUNK_S - 1)

    lanes = lax.iota(jnp.int32, LANES)
    cv = jnp.zeros((LANES,), jnp.float32)
    for b in range(B):
        cv = jnp.where(lanes == b, (cuts[b + 1] - cuts[b]).astype(jnp.float32), cv)
    cnt_v[...] = cv

    pltpu.sync_copy(acc_s, out_sum.at[wid])
    pltpu.sync_copy(acc_q, out_sq.at[wid])
    pltpu.sync_copy(cnt_v, out_cnt.at[wid])


_stats = pl.kernel(
    _stats_body,
    out_type=[
        jax.ShapeDtypeStruct((NW, B, C), jnp.float32),
        jax.ShapeDtypeStruct((NW, B, C), jnp.float32),
        jax.ShapeDtypeStruct((NW, B), jnp.float32),
    ],
    mesh=_mesh,
    scratch_types=[
        pltpu.VMEM((ROWS_WS,), jnp.int32),
        pltpu.VMEM((CHUNK, C), jnp.float32),
        pltpu.VMEM((CHUNK, C), jnp.float32),
        pltpu.VMEM((B, C), jnp.float32),
        pltpu.VMEM((B, C), jnp.float32),
        pltpu.VMEM((LANES,), jnp.float32),
        pltpu.SemaphoreType.DMA,
        pltpu.SemaphoreType.DMA,
    ],
    compiler_params=_sc_params,
)


def _combine_body(psum_ref, psq_ref, pcnt_ref,
                  gamma_ref, beta_ref, scale_ref, shift_ref):
    s = jnp.sum(psum_ref[...], axis=0)
    q = jnp.sum(psq_ref[...], axis=0)
    n = jnp.sum(pcnt_ref[...], axis=0)[:, None]
    nc = jnp.maximum(n, 1.0)
    mean = s / nc
    var = jnp.maximum(q / nc - mean * mean, 0.0)
    rstd = lax.rsqrt(var + EPS)
    scale = gamma_ref[...] * rstd
    scale_ref[...] = scale
    shift_ref[...] = beta_ref[...] - mean * scale


_combine = pl.pallas_call(
    _combine_body,
    out_shape=[
        jax.ShapeDtypeStruct((B, C), jnp.float32),
        jax.ShapeDtypeStruct((B, C), jnp.float32),
    ],
)


def _norm_body(feat_hbm, idx_hbm, scale_hbm, shift_hbm, out_hbm,
               idx_v, buf0, buf1, scale_v, shift_v, sem0, sem1, semo0, semo1):
    wid = lax.axis_index("s") * NC + lax.axis_index("c")
    row0 = wid * ROWS_W
    pltpu.sync_copy(idx_hbm.at[pl.ds(row0, ROWS_W)], idx_v)
    pltpu.sync_copy(scale_hbm, scale_v)
    pltpu.sync_copy(shift_hbm, shift_v)

    cuts = _segment_cuts(idx_v, ROWS_W)

    def out_copy(c, buf, sem):
        start = pl.multiple_of(row0 + c * CHUNK, 8)
        return pltpu.make_async_copy(buf, out_hbm.at[pl.ds(start, CHUNK)], sem)

    def process(buf, c):
        base = c * CHUNK
        for b in range(B):
            lo = jnp.clip(cuts[b] - base, 0, CHUNK)
            hi = jnp.clip(cuts[b + 1] - base, 0, CHUNK)

            @pl.when(hi > lo)
            def _(b=b, lo=lo, hi=hi):
                sc = [scale_v[b, pl.ds(j * LANES, LANES)] for j in range(CL)]
                sh = [shift_v[b, pl.ds(j * LANES, LANES)] for j in range(CL)]

                def rbody(r, _):
                    for j in range(CL):
                        sl = pl.ds(j * LANES, LANES)
                        buf[r, sl] = buf[r, sl] * sc[j] + sh[j]
                    return 0

                lax.fori_loop(lo, hi, rbody, 0)

    _in_copy(feat_hbm, row0, 0, buf0, sem0).start()
    _in_copy(feat_hbm, row0, 1, buf1, sem1).start()

    def pair(p, _):
        a = 2 * p
        _in_copy(feat_hbm, row0, 0, buf0, sem0).wait()
        process(buf0, a)
        out_copy(a, buf0, semo0).start()

        _in_copy(feat_hbm, row0, 0, buf1, sem1).wait()
        process(buf1, a + 1)
        out_copy(a + 1, buf1, semo1).start()

        out_copy(0, buf0, semo0).wait()

        @pl.when(a + 2 < NCHUNK)
        def _():
            _in_copy(feat_hbm, row0, a + 2, buf0, sem0).start()

        out_copy(0, buf1, semo1).wait()

        @pl.when(a + 3 < NCHUNK)
        def _():
            _in_copy(feat_hbm, row0, a + 3, buf1, sem1).start()

        return 0

    lax.fori_loop(0, NPAIR, pair, 0)
    if NCHUNK % 2:
        _in_copy(feat_hbm, row0, 0, buf0, sem0).wait()
        process(buf0, NCHUNK - 1)
        out_copy(NCHUNK - 1, buf0, semo0).start()
        out_copy(0, buf0, semo0).wait()


_norm = pl.kernel(
    _norm_body,
    out_type=jax.ShapeDtypeStruct((N, C), jnp.float32),
    mesh=_mesh,
    scratch_types=[
        pltpu.VMEM((ROWS_W,), jnp.int32),
        pltpu.VMEM((CHUNK, C), jnp.float32),
        pltpu.VMEM((CHUNK, C), jnp.float32),
        pltpu.VMEM((B, C), jnp.float32),
        pltpu.VMEM((B, C), jnp.float32),
        pltpu.SemaphoreType.DMA,
        pltpu.SemaphoreType.DMA,
        pltpu.SemaphoreType.DMA,
        pltpu.SemaphoreType.DMA,
    ],
    compiler_params=_sc_params,
)


def kernel(features, batch_indices, gamma, beta):
    idx = batch_indices.astype(jnp.int32)
    psum, psq, pcnt = _stats(features, idx)
    scale, shift = _combine(psum, psq, pcnt,
                            gamma.reshape(1, C), beta.reshape(1, C))
    return _norm(features, idx, scale, shift)


# speedup vs baseline: 1.1229x; 1.0192x over previous
import jax
import jax.numpy as jnp
from jax import lax
from jax.experimental import pallas as pl
from jax.experimental.pallas import tpu as pltpu
from jax.experimental.pallas import tpu_sc as plsc

N = 320000
C = 128
B = 16
EPS = 1e-5

NC = 2
NS = 16
NW = NC * NS
ROWS_W = N // NW
CHUNK = 400
NCHUNK = ROWS_W // CHUNK
NPAIR = NCHUNK // 2
LANES = 16
CL = C // LANES

ROWS_WS = ROWS_W
NCHUNK_S = ROWS_WS // CHUNK
NPAIR_S = NCHUNK_S // 2

_mesh = plsc.VectorSubcoreMesh(
    core_axis_name="c", subcore_axis_name="s", num_cores=NC, num_subcores=NS
)
_sc_params = pltpu.CompilerParams(needs_layout_passes=False)


def _segment_cuts(idx_v, n):
    bvec = lax.iota(jnp.int32, LANES)
    nn = jnp.full((LANES,), n, jnp.int32)

    def step(i, lo):
        st = jnp.int32(1 << 13) >> i
        cand = lo + st
        j = jnp.minimum(cand, nn) - 1
        val = plsc.load_gather(idx_v, [j])
        ok = (cand <= nn) & (val < bvec)
        return jnp.where(ok, cand, lo)

    lo = lax.fori_loop(0, 14, step, jnp.zeros((LANES,), jnp.int32))
    cuts = [lo[b] for b in range(LANES)]
    cuts.append(jnp.int32(n))
    return cuts


def _in_copy(feat_hbm, row0, c, buf, sem):
    start = pl.multiple_of(row0 + c * CHUNK, 8)
    return pltpu.make_async_copy(feat_hbm.at[pl.ds(start, CHUNK)], buf, sem)


def _stats_body(feat_hbm, idx_hbm, out_sum, out_sq, out_cnt,
                idx_v, buf0, buf1, acc_s, acc_q, cnt_v, sem0, sem1):
    wid = lax.axis_index("s") * NC + lax.axis_index("c")
    row0 = wid * ROWS_WS
    _in_copy(feat_hbm, row0, 0, buf0, sem0).start()
    _in_copy(feat_hbm, row0, 1, buf1, sem1).start()
    pltpu.sync_copy(idx_hbm.at[pl.ds(row0, ROWS_WS)], idx_v)

    zero = jnp.zeros((LANES,), jnp.float32)
    for b in range(B):
        for j in range(CL):
            acc_s[b, pl.ds(j * LANES, LANES)] = zero
            acc_q[b, pl.ds(j * LANES, LANES)] = zero

    cuts = _segment_cuts(idx_v, ROWS_WS)

    def process(buf, c):
        base = c * CHUNK
        for b in range(B):
            lo = jnp.clip(cuts[b] - base, 0, CHUNK)
            hi = jnp.clip(cuts[b + 1] - base, 0, CHUNK)

            @pl.when(hi > lo)
            def _(b=b, lo=lo, hi=hi):
                def rbody(r, carry):
                    ss = list(carry[:CL])
                    qq = list(carry[CL:])
                    for j in range(CL):
                        x = buf[r, pl.ds(j * LANES, LANES)]
                        ss[j] = ss[j] + x
                        qq[j] = qq[j] + x * x
                    return tuple(ss) + tuple(qq)

                res = lax.fori_loop(lo, hi, rbody, (zero,) * (2 * CL))
                for j in range(CL):
                    sl = pl.ds(j * LANES, LANES)
                    acc_s[b, sl] = acc_s[b, sl] + res[j]
                    acc_q[b, sl] = acc_q[b, sl] + res[CL + j]

    def pair(p, _):
        a = 2 * p
        _in_copy(feat_hbm, row0, 0, buf0, sem0).wait()
        process(buf0, a)

        @pl.when(a + 2 < NCHUNK_S)
        def _():
            _in_copy(feat_hbm, row0, a + 2, buf0, sem0).start()

        _in_copy(feat_hbm, row0, 0, buf1, sem1).wait()
        process(buf1, a + 1)

        @pl.when(a + 3 < NCHUNK_S)
        def _():
            _in_copy(feat_hbm, row0, a + 3, buf1, sem1).start()

        return 0

    lax.fori_loop(0, NPAIR_S, pair, 0)
    if NCHUNK_S % 2:
        _in_copy(feat_hbm, row0, 0, buf0, sem0).wait()
        process(buf0, NCHUNK_S - 1)

    lanes = lax.iota(jnp.int32, LANES)
    cv = jnp.zeros((LANES,), jnp.float32)
    for b in range(B):
        cv = jnp.where(lanes == b, (cuts[b + 1] - cuts[b]).astype(jnp.float32), cv)
    cnt_v[...] = cv

    pltpu.sync_copy(acc_s, out_sum.at[wid])
    pltpu.sync_copy(acc_q, out_sq.at[wid])
    pltpu.sync_copy(cnt_v, out_cnt.at[wid])


_stats = pl.kernel(
    _stats_body,
    out_type=[
        jax.ShapeDtypeStruct((NW, B, C), jnp.float32),
        jax.ShapeDtypeStruct((NW, B, C), jnp.float32),
        jax.ShapeDtypeStruct((NW, B), jnp.float32),
    ],
    mesh=_mesh,
    scratch_types=[
        pltpu.VMEM((ROWS_WS,), jnp.int32),
        pltpu.VMEM((CHUNK, C), jnp.float32),
        pltpu.VMEM((CHUNK, C), jnp.float32),
        pltpu.VMEM((B, C), jnp.float32),
        pltpu.VMEM((B, C), jnp.float32),
        pltpu.VMEM((LANES,), jnp.float32),
        pltpu.SemaphoreType.DMA,
        pltpu.SemaphoreType.DMA,
    ],
    compiler_params=_sc_params,
)


def _combine_body(psum_ref, psq_ref, pcnt_ref,
                  gamma_ref, beta_ref, scale_ref, shift_ref):
    s = jnp.sum(psum_ref[...], axis=0)
    q = jnp.sum(psq_ref[...], axis=0)
    n = jnp.sum(pcnt_ref[...], axis=0)[:, None]
    nc = jnp.maximum(n, 1.0)
    mean = s / nc
    var = jnp.maximum(q / nc - mean * mean, 0.0)
    rstd = lax.rsqrt(var + EPS)
    scale = gamma_ref[...] * rstd
    scale_ref[...] = scale
    shift_ref[...] = beta_ref[...] - mean * scale


_combine = pl.pallas_call(
    _combine_body,
    out_shape=[
        jax.ShapeDtypeStruct((B, C), jnp.float32),
        jax.ShapeDtypeStruct((B, C), jnp.float32),
    ],
)


def _norm_body(feat_hbm, idx_hbm, scale_hbm, shift_hbm, out_hbm,
               idx_v, buf0, buf1, scale_v, shift_v, sem0, sem1, semo0, semo1):
    wid = lax.axis_index("s") * NC + lax.axis_index("c")
    row0 = wid * ROWS_W
    _in_copy(feat_hbm, row0, 0, buf0, sem0).start()
    _in_copy(feat_hbm, row0, 1, buf1, sem1).start()
    pltpu.sync_copy(idx_hbm.at[pl.ds(row0, ROWS_W)], idx_v)
    pltpu.sync_copy(scale_hbm, scale_v)
    pltpu.sync_copy(shift_hbm, shift_v)

    cuts = _segment_cuts(idx_v, ROWS_W)

    def out_copy(c, buf, sem):
        start = pl.multiple_of(row0 + c * CHUNK, 8)
        return pltpu.make_async_copy(buf, out_hbm.at[pl.ds(start, CHUNK)], sem)

    def process(buf, c):
        base = c * CHUNK
        for b in range(B):
            lo = jnp.clip(cuts[b] - base, 0, CHUNK)
            hi = jnp.clip(cuts[b + 1] - base, 0, CHUNK)

            @pl.when(hi > lo)
            def _(b=b, lo=lo, hi=hi):
                sc = [scale_v[b, pl.ds(j * LANES, LANES)] for j in range(CL)]
                sh = [shift_v[b, pl.ds(j * LANES, LANES)] for j in range(CL)]

                def rbody(r, _):
                    for j in range(CL):
                        sl = pl.ds(j * LANES, LANES)
                        buf[r, sl] = buf[r, sl] * sc[j] + sh[j]
                    return 0

                lax.fori_loop(lo, hi, rbody, 0)

    def pair(p, _):
        a = 2 * p
        _in_copy(feat_hbm, row0, 0, buf0, sem0).wait()
        process(buf0, a)
        out_copy(a, buf0, semo0).start()

        _in_copy(feat_hbm, row0, 0, buf1, sem1).wait()
        process(buf1, a + 1)
        out_copy(a + 1, buf1, semo1).start()

        out_copy(0, buf0, semo0).wait()

        @pl.when(a + 2 < NCHUNK)
        def _():
            _in_copy(feat_hbm, row0, a + 2, buf0, sem0).start()

        out_copy(0, buf1, semo1).wait()

        @pl.when(a + 3 < NCHUNK)
        def _():
            _in_copy(feat_hbm, row0, a + 3, buf1, sem1).start()

        return 0

    lax.fori_loop(0, NPAIR, pair, 0)
    if NCHUNK % 2:
        _in_copy(feat_hbm, row0, 0, buf0, sem0).wait()
        process(buf0, NCHUNK - 1)
        out_copy(NCHUNK - 1, buf0, semo0).start()
        out_copy(0, buf0, semo0).wait()


_norm = pl.kernel(
    _norm_body,
    out_type=jax.ShapeDtypeStruct((N, C), jnp.float32),
    mesh=_mesh,
    scratch_types=[
        pltpu.VMEM((ROWS_W,), jnp.int32),
        pltpu.VMEM((CHUNK, C), jnp.float32),
        pltpu.VMEM((CHUNK, C), jnp.float32),
        pltpu.VMEM((B, C), jnp.float32),
        pltpu.VMEM((B, C), jnp.float32),
        pltpu.SemaphoreType.DMA,
        pltpu.SemaphoreType.DMA,
        pltpu.SemaphoreType.DMA,
        pltpu.SemaphoreType.DMA,
    ],
    compiler_params=_sc_params,
)


def kernel(features, batch_indices, gamma, beta):
    idx = batch_indices.astype(jnp.int32)
    psum, psq, pcnt = _stats(features, idx)
    scale, shift = _combine(psum, psq, pcnt,
                            gamma.reshape(1, C), beta.reshape(1, C))
    return _norm(features, idx, scale, shift)
